# baseline (speedup 1.0000x reference)
import jax
import jax.numpy as jnp
from jax.experimental import pallas as pl
from jax.experimental.pallas import tpu as pltpu

L, R, T, H, D = 4, 16, 2048, 8, 64
A = 4
HD = H * D
LR = L * R


def _patched_window(src_hbm, li, base, r, accept_ref, win_in, wsem):
    aligned = pl.multiple_of((base // 8) * 8, 8)
    off = base - aligned
    ld = pltpu.make_async_copy(
        src_hbm.at[pl.ds(li, 1), pl.ds(aligned, 16), :], win_in, wsem)
    ld.start()
    ld.wait()
    win = win_in[0, :, :]
    rows = jax.lax.broadcasted_iota(jnp.int32, (16, 1), 0)
    new = win
    for a in range(A):
        src = off + accept_ref[r, a]
        tgt = off + a
        shift = jax.lax.rem(tgt - src + 16, 16)
        rolled = pltpu.roll(win, shift, 0)
        new = jnp.where(rows == tgt, rolled, new)
    return aligned, new


def _copy_fix_kernel(cachelen_ref, accept_ref, k_hbm, v_hbm, out_ref,
                     win_in, win_out, sem, wsem):
    c = pl.program_id(0)
    i = pl.program_id(1)
    r = jax.lax.rem(i, R)
    base = cachelen_ref[r]

    def do(src_hbm):
        cp = pltpu.make_async_copy(
            src_hbm.at[pl.ds(i, 1), :, :], out_ref, sem)
        cp.start()
        aligned, new = _patched_window(
            src_hbm, i, base, r, accept_ref, win_in, wsem)
        win_out[0, :, :] = new
        cp.wait()
        st = pltpu.make_async_copy(
            win_out, out_ref.at[:, pl.ds(aligned, 16), :], wsem)
        st.start()
        st.wait()

    @pl.when(c == 0)
    def _():
        do(k_hbm)

    @pl.when(c == 1)
    def _():
        do(v_hbm)


def kernel(K_cache, V_cache, cachelen, accept_indices):
    Kr = jax.lax.bitcast_convert_type(K_cache, jnp.bfloat16).reshape(LR, T, HD)
    Vr = jax.lax.bitcast_convert_type(V_cache, jnp.bfloat16).reshape(LR, T, HD)
    grid_spec = pltpu.PrefetchScalarGridSpec(
        num_scalar_prefetch=2,
        grid=(2, LR),
        in_specs=[
            pl.BlockSpec(memory_space=pl.ANY),
            pl.BlockSpec(memory_space=pl.ANY),
        ],
        out_specs=pl.BlockSpec((1, T, HD), lambda c, i, cl, ai: (c * LR + i, 0, 0)),
        scratch_shapes=[
            pltpu.VMEM((1, 16, HD), jnp.bfloat16),
            pltpu.VMEM((1, 16, HD), jnp.bfloat16),
            pltpu.SemaphoreType.DMA,
            pltpu.SemaphoreType.DMA,
        ],
    )
    out = pl.pallas_call(
        _copy_fix_kernel,
        grid_spec=grid_spec,
        out_shape=jax.ShapeDtypeStruct((2 * LR, T, HD), jnp.bfloat16),
    )(cachelen, accept_indices, Kr, Vr)
    out = jax.lax.bitcast_convert_type(out, K_cache.dtype)
    return out.reshape(2, L, R, T, H, D)


# speedup vs baseline: 2.2349x; 2.2349x over previous
import jax
import jax.numpy as jnp
from jax.experimental import pallas as pl
from jax.experimental.pallas import tpu as pltpu

L, R, T, H, D = 4, 16, 2048, 8, 64
A = 4
HD = H * D
LR = L * R
WIN = 256
SUB = HD // 2
CHUNK = 64


def _copy_fix_kernel(cachelen_ref, accept_ref, k_ref, v_ref, out_ref):
    c = pl.program_id(0)
    i = pl.program_id(1)
    r = jax.lax.rem(i, R)
    base = cachelen_ref[r]
    start = jnp.minimum((base // 128) * 128, T - WIN)
    start = pl.multiple_of(start, 128)
    off = base - start

    def do(src_ref):
        s32 = src_ref.bitcast(jnp.int32)
        d32 = out_ref.bitcast(jnp.int32)
        d32[0, :, :] = s32[0, :, :]
        lanes = jax.lax.broadcasted_iota(jnp.int32, (CHUNK, WIN), 1)
        for ch in range(SUB // CHUNK):
            win = s32[0, pl.ds(ch * CHUNK, CHUNK), pl.ds(start, WIN)]
            new = win
            for a in range(A):
                src = off + accept_ref[r, a]
                tgt = off + a
                rolled = pltpu.roll(win, jax.lax.rem(tgt - src + WIN, WIN), 1)
                new = jnp.where(lanes == tgt, rolled, new)
            d32[0, pl.ds(ch * CHUNK, CHUNK), pl.ds(start, WIN)] = new

    @pl.when(c == 0)
    def _():
        do(k_ref)

    @pl.when(c == 1)
    def _():
        do(v_ref)


def kernel(K_cache, V_cache, cachelen, accept_indices):
    Kt = jnp.transpose(K_cache, (0, 1, 3, 4, 2)).reshape(LR, HD, T)
    Vt = jnp.transpose(V_cache, (0, 1, 3, 4, 2)).reshape(LR, HD, T)
    Kt = jax.lax.bitcast_convert_type(Kt, jnp.bfloat16)
    Vt = jax.lax.bitcast_convert_type(Vt, jnp.bfloat16)
    grid_spec = pltpu.PrefetchScalarGridSpec(
        num_scalar_prefetch=2,
        grid=(2, LR),
        in_specs=[
            pl.BlockSpec((1, HD, T), lambda c, i, cl, ai: (i * (1 - c), 0, 0)),
            pl.BlockSpec((1, HD, T), lambda c, i, cl, ai: (i * c, 0, 0)),
        ],
        out_specs=pl.BlockSpec((1, HD, T), lambda c, i, cl, ai: (c * LR + i, 0, 0)),
    )
    out = pl.pallas_call(
        _copy_fix_kernel,
        grid_spec=grid_spec,
        out_shape=jax.ShapeDtypeStruct((2 * LR, HD, T), jnp.bfloat16),
    )(cachelen, accept_indices, Kt, Vt)
    out = jax.lax.bitcast_convert_type(out, K_cache.dtype)
    out = out.reshape(2, L, R, H, D, T)
    return jnp.transpose(out, (0, 1, 2, 5, 3, 4))
